# baseline (speedup 1.0000x reference)
import jax
import jax.numpy as jnp
from jax import lax
from jax.experimental import pallas as pl
from jax.experimental.pallas import tpu as pltpu
from jax.experimental.pallas import tpu_sc as plsc

_K = 4
_L = 16
_NC = 2
_NS = 16
_NW = _NC * _NS
_NEG = -1.0e30


def _sort4(a, b, c, d):
    a, b = jnp.maximum(a, b), jnp.minimum(a, b)
    c, d = jnp.maximum(c, d), jnp.minimum(c, d)
    a, c = jnp.maximum(a, c), jnp.minimum(a, c)
    b, d = jnp.maximum(b, d), jnp.minimum(b, d)
    b, c = jnp.maximum(b, c), jnp.minimum(b, c)
    return a, b, c, d


def _merge44(ms, bs):
    a0, a1, a2, a3 = ms
    b0, b1, b2, b3 = bs
    m00 = jnp.minimum(a0, b0)
    m01 = jnp.minimum(a0, b1)
    m10 = jnp.minimum(a1, b0)
    m02 = jnp.minimum(a0, b2)
    m11 = jnp.minimum(a1, b1)
    m20 = jnp.minimum(a2, b0)
    c0 = jnp.maximum(a0, b0)
    c1 = jnp.maximum(m00, jnp.maximum(a1, b1))
    c2 = jnp.maximum(jnp.maximum(b2, a2), jnp.maximum(m01, m10))
    c3 = jnp.maximum(jnp.maximum(b3, a3),
                     jnp.maximum(m02, jnp.maximum(m11, m20)))
    return c0, c1, c2, c3


def _absorb4(ms, vs):
    return _merge44(ms, _sort4(*vs))


def _make_pool(n_b, n_c, n_c_sc, n_hw):
    b_per_w = n_b // _NW
    ct_per_b = n_c_sc // 128
    n_units = b_per_w * ct_per_b
    n_chunk = 4
    chunk_rows = n_hw // n_chunk
    n_tiles = chunk_rows // 8
    out_per_w = b_per_w * n_c_sc
    mesh = plsc.VectorSubcoreMesh(core_axis_name="c", subcore_axis_name="s")

    def body(y_hbm, w_hbm, out_hbm, wv, buf0, buf1, outv, sem0, sem1):
        cid = lax.axis_index("c")
        sid = lax.axis_index("s")
        wid = sid * _NC + cid
        row_base = wid * b_per_w * n_hw

        pltpu.sync_copy(w_hbm, wv)
        wr = [wv[t, :] for t in range(_K)]
        negv = jnp.full((_L,), _NEG, jnp.float32)
        bufs = (buf0, buf1)
        sems = (sem0, sem1)

        def src(row0, c0, q):
            r = pl.multiple_of(row0 + q * chunk_rows, chunk_rows)
            c = pl.multiple_of(c0, 128)
            return y_hbm.at[pl.ds(r, chunk_rows), pl.ds(c, 128)]

        def advance(row0, c0):
            wrap = c0 + 128 >= n_c_sc
            row0n = jnp.where(wrap, row0 + n_hw, row0)
            c0n = jnp.where(wrap, jnp.int32(0), c0 + 128)
            return row0n, c0n

        r00 = row_base + jnp.int32(0)
        c00 = jnp.int32(0)
        pltpu.async_copy(src(r00, c00, 0), buf0, sem0)
        pltpu.async_copy(src(r00, c00, 1), buf1, sem1)

        def unit_body(u, carry):
            row0, c0 = carry
            row0n, c0n = advance(row0, c0)
            ms = [(negv, negv, negv, negv) for _ in range(8)]
            for q in range(n_chunk):
                buf = bufs[q % 2]
                sem = sems[q % 2]
                pltpu.make_async_copy(src(r00, c00, 0), buf, sem).wait()
                for p in range(8):
                    def pass_body(t, m, _p=p, _buf=buf):
                        vs = []
                        for w8 in range(16):
                            vs.append(_buf[t * 16 + w8,
                                           pl.ds(_p * _L, _L)])
                            if len(vs) == 4:
                                m = _absorb4(m, tuple(vs))
                                vs = []
                        return m
                    ms[p] = lax.fori_loop(0, n_tiles // 2, pass_body, ms[p])
                nq = q + 2
                if nq < n_chunk:
                    pltpu.async_copy(src(row0, c0, nq), buf, sem)
                else:
                    @pl.when(u + 1 < n_units)
                    def _():
                        pltpu.async_copy(src(row0n, c0n, nq - n_chunk),
                                         buf, sem)
            obase = u * 128
            for p in range(8):
                m0, m1, m2, m3 = ms[p]
                res = m0 * wr[0] + m1 * wr[1] + m2 * wr[2] + m3 * wr[3]
                outv[pl.ds(obase + p * _L, _L)] = res
            return row0n, c0n

        lax.fori_loop(0, n_units, unit_body, (r00, c00))

        pltpu.sync_copy(outv, out_hbm.at[pl.ds(wid * out_per_w, out_per_w)])

    return pl.kernel(
        body,
        out_type=jax.ShapeDtypeStruct((n_b * n_c_sc,), jnp.float32),
        mesh=mesh,
        compiler_params=pltpu.CompilerParams(needs_layout_passes=False),
        scratch_types=[
            pltpu.VMEM((_K, _L), jnp.float32),
            pltpu.VMEM((chunk_rows, 128), jnp.float32),
            pltpu.VMEM((chunk_rows, 128), jnp.float32),
            pltpu.VMEM((out_per_w,), jnp.float32),
            pltpu.SemaphoreType.DMA,
            pltpu.SemaphoreType.DMA,
        ],
    )


def _tc_block(w_ref, y_ref, o_ref):
    n_hw = y_ref.shape[0]
    negv = jnp.full((8, y_ref.shape[1]), _NEG, jnp.float32)

    n_acc = 4
    rows_per_iter = 32 * n_acc

    def body(i, st):
        out = []
        for a in range(n_acc):
            base = i * rows_per_iter + a * 32
            vs = tuple(y_ref[pl.ds(base + t * 8, 8), :] for t in range(4))
            out.append(_absorb4(st[a], vs))
        return tuple(out)

    neg4 = (negv, negv, negv, negv)
    sts = lax.fori_loop(0, n_hw // rows_per_iter, body, (neg4,) * n_acc)
    while len(sts) > 1:
        sts = tuple(_merge44(sts[2 * i], sts[2 * i + 1])
                    for i in range(len(sts) // 2))
    ms = sts[0]
    for h in (4, 2, 1):
        a = tuple(m[:h] for m in ms)
        b = tuple(m[h:2 * h] for m in ms)
        ms = _merge44(a, b)
    acc = ms[0] * w_ref[0]
    for r in range(1, _K):
        acc = acc + ms[r] * w_ref[r]
    o_ref[...] = acc.reshape(o_ref.shape)


def _make_tc_pool(n_b_tc, n_c, n_hw, b_off):
    grid = (n_b_tc,)
    return pl.pallas_call(
        _tc_block,
        grid=grid,
        in_specs=[
            pl.BlockSpec(memory_space=pltpu.SMEM),
            pl.BlockSpec((n_hw, n_c), lambda i: (b_off + i, 0)),
        ],
        out_specs=pl.BlockSpec((1, 1, n_c), lambda i: (i, 0, 0)),
        out_shape=jax.ShapeDtypeStruct((n_b_tc, 1, n_c), jnp.float32),
        compiler_params=pltpu.CompilerParams(
            dimension_semantics=("arbitrary",)),
    )


_B_SC = 32


def kernel(x, weights):
    b, c, h, w = x.shape
    n_hw = h * w
    assert c % 128 == 0 and b % _NW == 0 and n_hw % 32 == 0
    y = x.transpose(0, 2, 3, 1).reshape(b * n_hw, c)
    wmat = jnp.broadcast_to(
        weights.reshape(_K, 1).astype(jnp.float32) / _K, (_K, _L))
    n_b_sc = _B_SC if 0 < _B_SC < b else b
    pool = _make_pool(n_b_sc, c, c, n_hw)
    out_sc = pool(y, wmat).reshape(n_b_sc, c)
    if n_b_sc < b:
        wvec = weights.reshape(_K).astype(jnp.float32) / _K
        tc_pool = _make_tc_pool(b - n_b_sc, c, n_hw, n_b_sc)
        out_tc = tc_pool(wvec, y).reshape(b - n_b_sc, c)
        out = jnp.concatenate([out_sc, out_tc], axis=0)
    else:
        out = out_sc
    return out.reshape(b, c, 1, 1)


# speedup vs baseline: 1.0669x; 1.0669x over previous
import jax
import jax.numpy as jnp
from jax import lax
from jax.experimental import pallas as pl
from jax.experimental.pallas import tpu as pltpu
from jax.experimental.pallas import tpu_sc as plsc

_K = 4
_L = 16
_NC = 2
_NS = 16
_NW = _NC * _NS
_NEG = -1.0e30


def _sort4(a, b, c, d):
    a, b = jnp.maximum(a, b), jnp.minimum(a, b)
    c, d = jnp.maximum(c, d), jnp.minimum(c, d)
    a, c = jnp.maximum(a, c), jnp.minimum(a, c)
    b, d = jnp.maximum(b, d), jnp.minimum(b, d)
    b, c = jnp.maximum(b, c), jnp.minimum(b, c)
    return a, b, c, d


def _merge44(ms, bs):
    a0, a1, a2, a3 = ms
    b0, b1, b2, b3 = bs
    m00 = jnp.minimum(a0, b0)
    m01 = jnp.minimum(a0, b1)
    m10 = jnp.minimum(a1, b0)
    m02 = jnp.minimum(a0, b2)
    m11 = jnp.minimum(a1, b1)
    m20 = jnp.minimum(a2, b0)
    c0 = jnp.maximum(a0, b0)
    c1 = jnp.maximum(m00, jnp.maximum(a1, b1))
    c2 = jnp.maximum(jnp.maximum(b2, a2), jnp.maximum(m01, m10))
    c3 = jnp.maximum(jnp.maximum(b3, a3),
                     jnp.maximum(m02, jnp.maximum(m11, m20)))
    return c0, c1, c2, c3


def _absorb4(ms, vs):
    return _merge44(ms, _sort4(*vs))


def _make_pool(n_b, n_c, n_c_sc, n_hw):
    b_per_w = n_b // _NW
    ct_per_b = n_c_sc // 128
    n_units = b_per_w * ct_per_b
    n_chunk = 4
    chunk_rows = n_hw // n_chunk
    n_tiles = chunk_rows // 8
    out_per_w = b_per_w * n_c_sc
    mesh = plsc.VectorSubcoreMesh(core_axis_name="c", subcore_axis_name="s")

    def body(y_hbm, w_hbm, out_hbm, wv, buf0, buf1, outv, sem0, sem1):
        cid = lax.axis_index("c")
        sid = lax.axis_index("s")
        wid = sid * _NC + cid
        row_base = wid * b_per_w * n_hw

        pltpu.sync_copy(w_hbm, wv)
        wr = [wv[t, :] for t in range(_K)]
        negv = jnp.full((_L,), _NEG, jnp.float32)
        bufs = (buf0, buf1)
        sems = (sem0, sem1)

        def src(row0, c0, q):
            r = pl.multiple_of(row0 + q * chunk_rows, chunk_rows)
            c = pl.multiple_of(c0, 128)
            return y_hbm.at[pl.ds(r, chunk_rows), pl.ds(c, 128)]

        def advance(row0, c0):
            wrap = c0 + 128 >= n_c_sc
            row0n = jnp.where(wrap, row0 + n_hw, row0)
            c0n = jnp.where(wrap, jnp.int32(0), c0 + 128)
            return row0n, c0n

        r00 = row_base + jnp.int32(0)
        c00 = jnp.int32(0)
        pltpu.async_copy(src(r00, c00, 0), buf0, sem0)
        pltpu.async_copy(src(r00, c00, 1), buf1, sem1)

        def unit_body(u, carry):
            row0, c0 = carry
            row0n, c0n = advance(row0, c0)
            ms = [(negv, negv, negv, negv) for _ in range(8)]
            for q in range(n_chunk):
                buf = bufs[q % 2]
                sem = sems[q % 2]
                pltpu.make_async_copy(src(r00, c00, 0), buf, sem).wait()
                for p in range(8):
                    def pass_body(t, m, _p=p, _buf=buf):
                        vs = []
                        for w8 in range(8):
                            vs.append(_buf[t * 8 + w8,
                                           pl.ds(_p * _L, _L)])
                            if len(vs) == 4:
                                m = _absorb4(m, tuple(vs))
                                vs = []
                        return m
                    ms[p] = lax.fori_loop(0, n_tiles, pass_body, ms[p])
                nq = q + 2
                if nq < n_chunk:
                    pltpu.async_copy(src(row0, c0, nq), buf, sem)
                else:
                    @pl.when(u + 1 < n_units)
                    def _():
                        pltpu.async_copy(src(row0n, c0n, nq - n_chunk),
                                         buf, sem)
            obase = u * 128
            for p in range(8):
                m0, m1, m2, m3 = ms[p]
                res = m0 * wr[0] + m1 * wr[1] + m2 * wr[2] + m3 * wr[3]
                outv[pl.ds(obase + p * _L, _L)] = res
            return row0n, c0n

        lax.fori_loop(0, n_units, unit_body, (r00, c00))

        pltpu.sync_copy(outv, out_hbm.at[pl.ds(wid * out_per_w, out_per_w)])

    return pl.kernel(
        body,
        out_type=jax.ShapeDtypeStruct((n_b * n_c_sc,), jnp.float32),
        mesh=mesh,
        compiler_params=pltpu.CompilerParams(needs_layout_passes=False),
        scratch_types=[
            pltpu.VMEM((_K, _L), jnp.float32),
            pltpu.VMEM((chunk_rows, 128), jnp.float32),
            pltpu.VMEM((chunk_rows, 128), jnp.float32),
            pltpu.VMEM((out_per_w,), jnp.float32),
            pltpu.SemaphoreType.DMA,
            pltpu.SemaphoreType.DMA,
        ],
    )


def _tc_block(w_ref, y_ref, o_ref):
    n_hw = y_ref.shape[0]
    negv = jnp.full((8, y_ref.shape[1]), _NEG, jnp.float32)

    n_acc = 4
    rows_per_iter = 32 * n_acc

    def body(i, st):
        out = []
        for a in range(n_acc):
            base = i * rows_per_iter + a * 32
            vs = tuple(y_ref[pl.ds(base + t * 8, 8), :] for t in range(4))
            out.append(_absorb4(st[a], vs))
        return tuple(out)

    neg4 = (negv, negv, negv, negv)
    sts = lax.fori_loop(0, n_hw // rows_per_iter, body, (neg4,) * n_acc)
    while len(sts) > 1:
        sts = tuple(_merge44(sts[2 * i], sts[2 * i + 1])
                    for i in range(len(sts) // 2))
    ms = sts[0]
    for h in (4, 2, 1):
        a = tuple(m[:h] for m in ms)
        b = tuple(m[h:2 * h] for m in ms)
        ms = _merge44(a, b)
    acc = ms[0] * w_ref[0]
    for r in range(1, _K):
        acc = acc + ms[r] * w_ref[r]
    o_ref[...] = acc.reshape(o_ref.shape)


def _make_tc_pool(n_b_tc, n_c, n_hw, b_off):
    grid = (n_b_tc,)
    return pl.pallas_call(
        _tc_block,
        grid=grid,
        in_specs=[
            pl.BlockSpec(memory_space=pltpu.SMEM),
            pl.BlockSpec((n_hw, n_c), lambda i: (b_off + i, 0)),
        ],
        out_specs=pl.BlockSpec((1, 1, n_c), lambda i: (i, 0, 0)),
        out_shape=jax.ShapeDtypeStruct((n_b_tc, 1, n_c), jnp.float32),
        compiler_params=pltpu.CompilerParams(
            dimension_semantics=("arbitrary",)),
    )


_B_SC = 32


def kernel(x, weights):
    b, c, h, w = x.shape
    n_hw = h * w
    assert c % 128 == 0 and b % _NW == 0 and n_hw % 32 == 0
    y = x.transpose(0, 2, 3, 1).reshape(b * n_hw, c)
    wmat = jnp.broadcast_to(
        weights.reshape(_K, 1).astype(jnp.float32) / _K, (_K, _L))
    n_b_sc = _B_SC if 0 < _B_SC < b else b
    pool = _make_pool(n_b_sc, c, c, n_hw)
    out_sc = pool(y, wmat).reshape(n_b_sc, c)
    if n_b_sc < b:
        wvec = weights.reshape(_K).astype(jnp.float32) / _K
        tc_pool = _make_tc_pool(b - n_b_sc, c, n_hw, n_b_sc)
        out_tc = tc_pool(wvec, y).reshape(b - n_b_sc, c)
        out = jnp.concatenate([out_sc, out_tc], axis=0)
    else:
        out = out_sc
    return out.reshape(b, c, 1, 1)
